# Initial kernel scaffold; baseline (speedup 1.0000x reference)
import jax
import jax.numpy as jnp
from jax.experimental import pallas as pl


def kernel(word_table, pos_table, input_ids, position_ids):
    raise NotImplementedError("write your pallas kernel here")


# speedup vs baseline: 3.0326x; 3.0326x over previous
import jax
import jax.numpy as jnp
from jax.experimental import pallas as pl
from jax.experimental.pallas import tpu as pltpu

_UNROLL = 16


def _gather_add_kernel(wid_ref, pid_ref, wtab_ref, ptab_ref, out_ref):
    tm = out_ref.shape[0]

    def chunk(c, carry):
        base = c * _UNROLL
        for u in range(_UNROLL):
            t = base + u
            wi = wid_ref[0, 0, t]
            pi = pid_ref[0, 0, t]
            out_ref[t, 0] = wtab_ref[wi, 0] + ptab_ref[pi, 0]
        return carry

    jax.lax.fori_loop(0, tm // _UNROLL, chunk, 0)


def _round_up(x: int, m: int) -> int:
    return ((x + m - 1) // m) * m


def seq_gnn_node_embedding_fast(word_table, pos_table, input_ids,
                                position_ids=None, *, add_position=True,
                                block_tm=1024):
    vocab, dim = word_table.shape
    orig_shape = input_ids.shape

    flat_w = input_ids.reshape(-1).astype(jnp.int32)
    n = flat_w.shape[0]
    if n == 0:
        return jnp.zeros(orig_shape + (dim,), dtype=word_table.dtype)

    use_pos = add_position and (position_ids is not None)

    tm = max(_UNROLL, min(block_tm, _round_up(n, _UNROLL)))
    n_pad = _round_up(n, tm)
    pad = n_pad - n
    n_blocks = n_pad // tm

    w_ids = jnp.pad(flat_w, (0, pad)).reshape(n_blocks, 1, tm)

    max_pos = pos_table.shape[0]
    if use_pos:
        flat_p = jnp.minimum(position_ids.reshape(-1).astype(jnp.int32),
                             max_pos - 1)
        p_ids = jnp.pad(flat_p, (0, pad)).reshape(n_blocks, 1, tm)
        ptab3 = pos_table.reshape(max_pos, 1, dim)
    else:
        p_ids = jnp.zeros((n_blocks, 1, tm), dtype=jnp.int32)
        ptab3 = jnp.zeros((1, 1, dim), dtype=word_table.dtype)
        max_pos = 1

    wtab3 = word_table.reshape(vocab, 1, dim)

    grid = (n_blocks,)
    out = pl.pallas_call(
        _gather_add_kernel,
        out_shape=jax.ShapeDtypeStruct((n_pad, 1, dim), word_table.dtype),
        grid=grid,
        in_specs=[
            pl.BlockSpec((1, 1, tm), lambda i: (i, 0, 0),
                         memory_space=pltpu.SMEM),
            pl.BlockSpec((1, 1, tm), lambda i: (i, 0, 0),
                         memory_space=pltpu.SMEM),
            pl.BlockSpec((vocab, 1, dim), lambda i: (0, 0, 0)),
            pl.BlockSpec((max_pos, 1, dim), lambda i: (0, 0, 0)),
        ],
        out_specs=pl.BlockSpec((tm, 1, dim), lambda i: (i, 0, 0)),
        compiler_params=pltpu.CompilerParams(
            dimension_semantics=("parallel",),
            vmem_limit_bytes=60 * 1024 * 1024,
        ),
    )(w_ids, p_ids, wtab3, ptab3)

    return out[:n, 0].reshape(orig_shape + (dim,))


def kernel(word_table, pos_table, input_ids, position_ids):
    return seq_gnn_node_embedding_fast(word_table, pos_table, input_ids,
                                       position_ids)


# speedup vs baseline: 3.2131x; 1.0595x over previous
import jax
import jax.numpy as jnp
from jax.experimental import pallas as pl
from jax.experimental.pallas import tpu as pltpu

_UNROLL = 32


def _gather_add_kernel(wid_ref, pid_ref, wtab_ref, ptab_ref, out_ref):
    tm = out_ref.shape[0]

    def chunk(c, carry):
        base = c * _UNROLL
        for u in range(_UNROLL):
            t = base + u
            wi = wid_ref[0, 0, t]
            pi = pid_ref[0, 0, t]
            out_ref[t, 0] = wtab_ref[wi, 0] + ptab_ref[pi, 0]
        return carry

    jax.lax.fori_loop(0, tm // _UNROLL, chunk, 0)


def _round_up(x: int, m: int) -> int:
    return ((x + m - 1) // m) * m


def seq_gnn_node_embedding_fast(word_table, pos_table, input_ids,
                                position_ids=None, *, add_position=True,
                                block_tm=2048):
    vocab, dim = word_table.shape
    orig_shape = input_ids.shape

    flat_w = input_ids.reshape(-1).astype(jnp.int32)
    n = flat_w.shape[0]
    if n == 0:
        return jnp.zeros(orig_shape + (dim,), dtype=word_table.dtype)

    use_pos = add_position and (position_ids is not None)

    tm = max(_UNROLL, min(block_tm, _round_up(n, _UNROLL)))
    n_pad = _round_up(n, tm)
    pad = n_pad - n
    n_blocks = n_pad // tm

    w_ids = jnp.pad(flat_w, (0, pad)).reshape(n_blocks, 1, tm)

    max_pos = pos_table.shape[0]
    if use_pos:
        flat_p = jnp.minimum(position_ids.reshape(-1).astype(jnp.int32),
                             max_pos - 1)
        p_ids = jnp.pad(flat_p, (0, pad)).reshape(n_blocks, 1, tm)
        ptab3 = pos_table.reshape(max_pos, 1, dim)
    else:
        p_ids = jnp.zeros((n_blocks, 1, tm), dtype=jnp.int32)
        ptab3 = jnp.zeros((1, 1, dim), dtype=word_table.dtype)
        max_pos = 1

    wtab3 = word_table.reshape(vocab, 1, dim)

    grid = (n_blocks,)
    out = pl.pallas_call(
        _gather_add_kernel,
        out_shape=jax.ShapeDtypeStruct((n_pad, 1, dim), word_table.dtype),
        grid=grid,
        in_specs=[
            pl.BlockSpec((1, 1, tm), lambda i: (i, 0, 0),
                         memory_space=pltpu.SMEM),
            pl.BlockSpec((1, 1, tm), lambda i: (i, 0, 0),
                         memory_space=pltpu.SMEM),
            pl.BlockSpec((vocab, 1, dim), lambda i: (0, 0, 0)),
            pl.BlockSpec((max_pos, 1, dim), lambda i: (0, 0, 0)),
        ],
        out_specs=pl.BlockSpec((tm, 1, dim), lambda i: (i, 0, 0)),
        compiler_params=pltpu.CompilerParams(
            dimension_semantics=("arbitrary",),
            vmem_limit_bytes=60 * 1024 * 1024,
        ),
    )(w_ids, p_ids, wtab3, ptab3)

    return out[:n, 0].reshape(orig_shape + (dim,))


def kernel(word_table, pos_table, input_ids, position_ids):
    return seq_gnn_node_embedding_fast(word_table, pos_table, input_ids,
                                       position_ids)
